# Initial kernel scaffold; baseline (speedup 1.0000x reference)
import jax
import jax.numpy as jnp
from jax.experimental import pallas as pl


def kernel(x, adj_matrix, W1, att_src1, att_dst1, bias1, ln_w, ln_b, W2, att_src2, att_dst2, bias2, prob):
    raise NotImplementedError("write your pallas kernel here")


# speedup vs baseline: 3800.7812x; 3800.7812x over previous
import jax
import jax.numpy as jnp
from jax.experimental import pallas as pl

_BJ = 256
_PREC = jax.lax.Precision.HIGHEST


def _dot(a, b, dims):
    return jax.lax.dot_general(a, b, (dims, ((), ())),
                               preferred_element_type=jnp.float32,
                               precision=_PREC)


def _prep_kernel(heads, hid, adj_ref, x_ref, w1_ref, as1_ref, ad1_ref,
                 na_ref, nw_ref, ndeg_ref, er_ref, h1_ref, asrc_ref, ad1t_ref):
    a = adj_ref[:]
    ii = jax.lax.broadcasted_iota(jnp.int32, a.shape, 0)
    jj = jax.lax.broadcasted_iota(jnp.int32, a.shape, 1)
    api = a + (ii == jj).astype(jnp.float32)
    deg = jnp.sum(api, axis=1, keepdims=True)
    dcol = jax.lax.rsqrt(deg)
    drow = jnp.transpose(dcol, (1, 0))
    na = api * dcol * drow
    na_ref[:] = na
    nw_ref[:] = jnp.sum(na)[None, None]
    ndeg_ref[:] = jnp.sum((api != 0).astype(jnp.float32), axis=1,
                          keepdims=True) + 1.0
    cnt = jnp.sum((a != 0).astype(jnp.float32))
    er_ref[:] = (cnt / jnp.sum(a))[None, None]
    h1 = _dot(x_ref[:], w1_ref[:], ((1,), (0,)))
    h1_ref[:] = h1
    cols_s, rows_d = [], []
    for h in range(heads):
        h1h = h1[:, h * hid:(h + 1) * hid]
        cols_s.append(jnp.sum(h1h * as1_ref[h:h + 1, :], axis=1, keepdims=True))
        rows_d.append(_dot(ad1_ref[h:h + 1, :], h1h, ((1,), (1,))))
    asrc_ref[:] = jnp.concatenate(cols_s, axis=1)
    ad1t_ref[:] = jnp.concatenate(rows_d, axis=0)


def _masked_softmax_cols(mask, asrc_col, adst_row):
    logit = asrc_col + adst_row
    z = jnp.maximum(logit, 0.2 * logit)
    z = jnp.where(mask, z, -jnp.inf)
    amax = jnp.max(z, axis=0, keepdims=True)
    ex = jnp.exp(z - amax)
    den = jnp.sum(ex, axis=0, keepdims=True) + 1e-16
    return ex / den


def _att1_kernel(heads, hid, ncls, na_ref, h1_ref, asrc_ref, ad1t_ref, b1_ref,
                 lnw_ref, lnb_ref, w2_ref, as2_ref, ad2_ref,
                 h2_ref, asrc2_ref, ad2t_ref):
    mask = na_ref[:] != 0
    outs = []
    for h in range(heads):
        s = _masked_softmax_cols(mask, asrc_ref[:, h:h + 1], ad1t_ref[h:h + 1, :])
        outs.append(_dot(s, h1_ref[:, h * hid:(h + 1) * hid], ((0,), (0,))))
    hcat = jnp.concatenate(outs, axis=1) + b1_ref[:]
    hcat = jnp.where(hcat > 0, hcat, jnp.exp(jnp.minimum(hcat, 0.0)) - 1.0)
    mu = jnp.mean(hcat, axis=1, keepdims=True)
    var = jnp.mean((hcat - mu) ** 2, axis=1, keepdims=True)
    hm = (hcat - mu) / jnp.sqrt(var + 1e-5) * lnw_ref[:] + lnb_ref[:]
    h2 = _dot(hm, w2_ref[:], ((1,), (0,)))
    h2_ref[:] = h2
    cols_s, rows_d = [], []
    for h in range(heads):
        h2h = h2[:, h * ncls:(h + 1) * ncls]
        cols_s.append(jnp.sum(h2h * as2_ref[h:h + 1, :], axis=1, keepdims=True))
        rows_d.append(_dot(ad2_ref[h:h + 1, :], h2h, ((1,), (1,))))
    asrc2_ref[:] = jnp.concatenate(cols_s, axis=1)
    ad2t_ref[:] = jnp.concatenate(rows_d, axis=0)


def _att2_kernel(heads, ncls, na_ref, h2_ref, asrc_ref, ad2t_ref, b2_ref,
                 emb_ref, logp_ref, s0_ref):
    mask = na_ref[:] != 0
    acc = None
    for h in range(heads):
        s = _masked_softmax_cols(mask, asrc_ref[:, h:h + 1], ad2t_ref[h:h + 1, :])
        if h == 0:
            s0_ref[:] = s
        o = _dot(s, h2_ref[:, h * ncls:(h + 1) * ncls], ((0,), (0,)))
        acc = o if acc is None else acc + o
    hout = acc * (1.0 / heads) + b2_ref[:]
    emb_ref[:] = hout
    m = jnp.max(hout, axis=1, keepdims=True)
    sh = hout - m
    logp_ref[:] = sh - jnp.log(jnp.sum(jnp.exp(sh), axis=1, keepdims=True))


def _dirichlet_kernel(bj, s0_ref, emb_ref, ndeg_ref, nw_ref, acc_ref):
    jb = pl.program_id(0)
    u = jnp.maximum(emb_ref[:], 0.0) * jax.lax.rsqrt(ndeg_ref[:])
    p = jnp.sum(u * u, axis=1, keepdims=True)
    s = s0_ref[:]
    t_u = _dot(s, u, ((0,), (0,)))
    t_p = _dot(s, p, ((0,), (0,)))
    colsum = jnp.sum(s, axis=0, keepdims=True)
    emb_blk = emb_ref[pl.ds(jb * bj, bj), :]
    ndeg_blk = ndeg_ref[pl.ds(jb * bj, bj), :]
    u_blk = jnp.maximum(emb_blk, 0.0) * jax.lax.rsqrt(ndeg_blk)
    p_blk = jnp.sum(u_blk * u_blk, axis=1, keepdims=True)
    term_q = _dot(colsum, p_blk, ((1,), (0,)))[0, 0]
    partial = jnp.sum(t_p) + term_q - 2.0 * jnp.sum(u_blk * t_u)

    @pl.when(jb == 0)
    def _():
        acc_ref[:] = jnp.zeros((1, 1), jnp.float32)

    acc_ref[:] += partial[None, None]

    @pl.when(jb == pl.num_programs(0) - 1)
    def _():
        nw = nw_ref[:]
        de = acc_ref[:] * 0.5
        acc_ref[:] = jnp.where(nw != 0.0, de / nw, jnp.zeros((1, 1), jnp.float32))


def kernel(x, adj_matrix, W1, att_src1, att_dst1, bias1, ln_w, ln_b, W2,
           att_src2, att_dst2, bias2, prob):
    n, f_in = x.shape
    heads, hid = att_src1.shape
    ncls = att_src2.shape[1]
    fmid = heads * hid
    nb = n // _BJ
    f32 = jnp.float32

    b1r = bias1.reshape(1, fmid)
    lnwr = ln_w.reshape(1, fmid)
    lnbr = ln_b.reshape(1, fmid)
    b2r = bias2.reshape(1, ncls)

    full = lambda shape: pl.BlockSpec(shape, lambda j: (0,) * len(shape))

    na, nw, ndeg, er, h1, asrc1, ad1t = pl.pallas_call(
        lambda *refs: _prep_kernel(heads, hid, *refs),
        out_shape=(
            jax.ShapeDtypeStruct((n, n), f32),
            jax.ShapeDtypeStruct((1, 1), f32),
            jax.ShapeDtypeStruct((n, 1), f32),
            jax.ShapeDtypeStruct((1, 1), f32),
            jax.ShapeDtypeStruct((n, fmid), f32),
            jax.ShapeDtypeStruct((n, heads), f32),
            jax.ShapeDtypeStruct((heads, n), f32),
        ),
    )(adj_matrix, x, W1, att_src1, att_dst1)

    h2, asrc2, ad2t = pl.pallas_call(
        lambda *refs: _att1_kernel(heads, hid, ncls, *refs),
        grid=(nb,),
        in_specs=[
            pl.BlockSpec((n, _BJ), lambda j: (0, j)),
            full((n, fmid)),
            full((n, heads)),
            pl.BlockSpec((heads, _BJ), lambda j: (0, j)),
            full((1, fmid)),
            full((1, fmid)),
            full((1, fmid)),
            full((fmid, heads * ncls)),
            full((heads, ncls)),
            full((heads, ncls)),
        ],
        out_specs=(
            pl.BlockSpec((_BJ, heads * ncls), lambda j: (j, 0)),
            pl.BlockSpec((_BJ, heads), lambda j: (j, 0)),
            pl.BlockSpec((heads, _BJ), lambda j: (0, j)),
        ),
        out_shape=(
            jax.ShapeDtypeStruct((n, heads * ncls), f32),
            jax.ShapeDtypeStruct((n, heads), f32),
            jax.ShapeDtypeStruct((heads, n), f32),
        ),
    )(na, h1, asrc1, ad1t, b1r, lnwr, lnbr, W2, att_src2, att_dst2)

    emb, logp, s0 = pl.pallas_call(
        lambda *refs: _att2_kernel(heads, ncls, *refs),
        grid=(nb,),
        in_specs=[
            pl.BlockSpec((n, _BJ), lambda j: (0, j)),
            full((n, heads * ncls)),
            full((n, heads)),
            pl.BlockSpec((heads, _BJ), lambda j: (0, j)),
            full((1, ncls)),
        ],
        out_specs=(
            pl.BlockSpec((_BJ, ncls), lambda j: (j, 0)),
            pl.BlockSpec((_BJ, ncls), lambda j: (j, 0)),
            pl.BlockSpec((n, _BJ), lambda j: (0, j)),
        ),
        out_shape=(
            jax.ShapeDtypeStruct((n, ncls), f32),
            jax.ShapeDtypeStruct((n, ncls), f32),
            jax.ShapeDtypeStruct((n, n), f32),
        ),
    )(na, h2, asrc2, ad2t, b2r)

    de = pl.pallas_call(
        lambda *refs: _dirichlet_kernel(_BJ, *refs),
        grid=(nb,),
        in_specs=[
            pl.BlockSpec((n, _BJ), lambda j: (0, j)),
            full((n, ncls)),
            full((n, 1)),
            full((1, 1)),
        ],
        out_specs=pl.BlockSpec((1, 1), lambda j: (0, 0)),
        out_shape=jax.ShapeDtypeStruct((1, 1), f32),
    )(s0, emb, ndeg, nw)

    return (emb, logp, de.reshape(()), prob, er.reshape(()), na)
